# baseline (speedup 1.0000x reference)
import jax
import jax.numpy as jnp
from jax import lax
from jax.experimental import pallas as pl
from jax.experimental.pallas import tpu as pltpu
from jax.experimental.pallas import tpu_sc as plsc

_NUM_VIEWS = 8
_PROMPT_LEN = 50
_DIM = 768
_BATCH = 1024
_PLEN_PAD = 56
_PLEN_SC = 32
_TAIL = _PROMPT_LEN - _PLEN_SC

_info = plsc.get_sparse_core_info()
_NC, _NS = _info.num_cores, _info.num_subcores
_NW = _NC * _NS
_BPW = _BATCH // _NW
_NBUF = 3
_BB = 64


def _sc_gather(view_id, table):
    mesh = plsc.VectorSubcoreMesh(core_axis_name="c", subcore_axis_name="s")

    @pl.kernel(
        mesh=mesh,
        out_type=jax.ShapeDtypeStruct((_BATCH, _PROMPT_LEN, _DIM), jnp.float32),
        scratch_types=[
            pltpu.VMEM((_BPW, 1), jnp.int32),
            pltpu.VMEM((1, _PLEN_SC, _DIM), jnp.float32),
            pltpu.VMEM((1, _PLEN_SC, _DIM), jnp.float32),
            pltpu.VMEM((1, _PLEN_SC, _DIM), jnp.float32),
            pltpu.SemaphoreType.DMA,
            pltpu.SemaphoreType.DMA,
            pltpu.SemaphoreType.DMA,
            pltpu.SemaphoreType.DMA,
            pltpu.SemaphoreType.DMA,
            pltpu.SemaphoreType.DMA,
        ],
    )
    def k(idx_hbm, table_hbm, out_hbm, idx_v, b0, b1, b2, g0, g1, g2, s0, s1, s2):
        wid = lax.axis_index("s") * _NC + lax.axis_index("c")
        base = wid * _BPW
        bufs, gsems, ssems = (b0, b1, b2), (g0, g1, g2), (s0, s1, s2)

        def start_gather(j, b):
            pltpu.make_async_copy(
                table_hbm.at[idx_v.at[j]], bufs[b], gsems[b]
            ).start()

        def wait_gather(b):
            pltpu.make_async_copy(
                table_hbm.at[pl.ds(0, 1)], bufs[b], gsems[b]
            ).wait()

        def start_scatter(j, b):
            pltpu.make_async_copy(
                bufs[b],
                out_hbm.at[pl.ds(base + j, 1), pl.ds(0, _PLEN_SC), :],
                ssems[b],
            ).start()

        def wait_scatter(b):
            pltpu.make_async_copy(
                bufs[b],
                out_hbm.at[pl.ds(base, 1), pl.ds(0, _PLEN_SC), :],
                ssems[b],
            ).wait()

        pltpu.sync_copy(idx_hbm.at[pl.ds(base, _BPW)], idx_v)
        for b in range(_NBUF):
            start_gather(b, b)

        def body(i, _):
            for b in range(_NBUF):
                j = _NBUF * i + b
                wait_gather(b)
                start_scatter(j, b)
            for b in range(_NBUF):
                j = _NBUF * i + b
                wait_scatter(b)

                @pl.when(j + _NBUF < _BPW)
                def _():
                    start_gather(j + _NBUF, b)

            return ()

        lax.fori_loop(0, _BPW // _NBUF, body, (), unroll=False)
        for j in range((_BPW // _NBUF) * _NBUF, _BPW):
            b = j % _NBUF
            wait_gather(b)
            start_scatter(j, b)
            wait_scatter(b)

    return k(view_id.reshape(_BATCH, 1), table)


def _tc_tail(view_id, prompts, sc_out):
    def body(idx_ref, tbl_ref, _aliased_ref, out_ref, stage, sem):
        i = pl.program_id(0)
        for r in range(_BB):
            v = idx_ref[i * _BB + r]
            stage[r] = tbl_ref[v, pl.ds(_PLEN_SC, _TAIL), :]
        copy = pltpu.make_async_copy(
            stage,
            out_ref.at[pl.ds(i * _BB, _BB), pl.ds(_PLEN_SC, _TAIL), :],
            sem,
        )
        copy.start()
        copy.wait()

    return pl.pallas_call(
        body,
        grid_spec=pltpu.PrefetchScalarGridSpec(
            num_scalar_prefetch=1,
            grid=(_BATCH // _BB,),
            in_specs=[
                pl.BlockSpec((_NUM_VIEWS, _PROMPT_LEN, _DIM), lambda i, idx: (0, 0, 0)),
                pl.BlockSpec(memory_space=pltpu.HBM),
            ],
            out_specs=pl.BlockSpec(memory_space=pltpu.HBM),
            scratch_shapes=[
                pltpu.VMEM((_BB, _TAIL, _DIM), jnp.float32),
                pltpu.SemaphoreType.DMA,
            ],
        ),
        out_shape=jax.ShapeDtypeStruct((_BATCH, _PROMPT_LEN, _DIM), jnp.float32),
        input_output_aliases={2: 0},
    )(view_id, prompts, sc_out)


def kernel(view_id, prompts):
    idx = view_id.astype(jnp.int32)
    table_sc = prompts[:, :_PLEN_SC, :]
    sc_out = _sc_gather(idx, table_sc)
    return _tc_tail(idx, prompts, sc_out)


# speedup vs baseline: 1.0814x; 1.0814x over previous
import jax
import jax.numpy as jnp
from jax import lax
from jax.experimental import pallas as pl
from jax.experimental.pallas import tpu as pltpu
from jax.experimental.pallas import tpu_sc as plsc

_NUM_VIEWS = 8
_PROMPT_LEN = 50
_DIM = 768
_BATCH = 1024
_PLEN_PAD = 56
_PLEN_SC = 24
_TAIL = _PROMPT_LEN - _PLEN_SC

_info = plsc.get_sparse_core_info()
_NC, _NS = _info.num_cores, _info.num_subcores
_NW = _NC * _NS
_BPW = _BATCH // _NW
_NBUF = 3
_BB = 64


def _sc_gather(view_id, table):
    mesh = plsc.VectorSubcoreMesh(core_axis_name="c", subcore_axis_name="s")

    @pl.kernel(
        mesh=mesh,
        out_type=jax.ShapeDtypeStruct((_BATCH, _PROMPT_LEN, _DIM), jnp.float32),
        scratch_types=[
            pltpu.VMEM((_BPW, 1), jnp.int32),
            pltpu.VMEM((1, _PLEN_SC, _DIM), jnp.float32),
            pltpu.VMEM((1, _PLEN_SC, _DIM), jnp.float32),
            pltpu.VMEM((1, _PLEN_SC, _DIM), jnp.float32),
            pltpu.SemaphoreType.DMA,
            pltpu.SemaphoreType.DMA,
            pltpu.SemaphoreType.DMA,
            pltpu.SemaphoreType.DMA,
            pltpu.SemaphoreType.DMA,
            pltpu.SemaphoreType.DMA,
        ],
    )
    def k(idx_hbm, table_hbm, out_hbm, idx_v, b0, b1, b2, g0, g1, g2, s0, s1, s2):
        wid = lax.axis_index("s") * _NC + lax.axis_index("c")
        base = wid * _BPW
        bufs, gsems, ssems = (b0, b1, b2), (g0, g1, g2), (s0, s1, s2)

        def start_gather(j, b):
            pltpu.make_async_copy(
                table_hbm.at[idx_v.at[j]], bufs[b], gsems[b]
            ).start()

        def wait_gather(b):
            pltpu.make_async_copy(
                table_hbm.at[pl.ds(0, 1)], bufs[b], gsems[b]
            ).wait()

        def start_scatter(j, b):
            pltpu.make_async_copy(
                bufs[b],
                out_hbm.at[pl.ds(base + j, 1), pl.ds(0, _PLEN_SC), :],
                ssems[b],
            ).start()

        def wait_scatter(b):
            pltpu.make_async_copy(
                bufs[b],
                out_hbm.at[pl.ds(base, 1), pl.ds(0, _PLEN_SC), :],
                ssems[b],
            ).wait()

        pltpu.sync_copy(idx_hbm.at[pl.ds(base, _BPW)], idx_v)
        for b in range(_NBUF):
            start_gather(b, b)

        def body(i, _):
            for b in range(_NBUF):
                j = _NBUF * i + b
                wait_gather(b)
                start_scatter(j, b)
            for b in range(_NBUF):
                j = _NBUF * i + b
                wait_scatter(b)

                @pl.when(j + _NBUF < _BPW)
                def _():
                    start_gather(j + _NBUF, b)

            return ()

        lax.fori_loop(0, _BPW // _NBUF, body, (), unroll=False)
        for j in range((_BPW // _NBUF) * _NBUF, _BPW):
            b = j % _NBUF
            wait_gather(b)
            start_scatter(j, b)
            wait_scatter(b)

    return k(view_id.reshape(_BATCH, 1), table)


def _tc_tail(view_id, prompts, sc_out):
    def body(idx_ref, tbl_ref, _aliased_ref, out_ref, stage, sem):
        i = pl.program_id(0)
        for r in range(_BB):
            v = idx_ref[i * _BB + r]
            stage[r] = tbl_ref[v, pl.ds(_PLEN_SC, _TAIL), :]
        copy = pltpu.make_async_copy(
            stage,
            out_ref.at[pl.ds(i * _BB, _BB), pl.ds(_PLEN_SC, _TAIL), :],
            sem,
        )
        copy.start()
        copy.wait()

    return pl.pallas_call(
        body,
        grid_spec=pltpu.PrefetchScalarGridSpec(
            num_scalar_prefetch=1,
            grid=(_BATCH // _BB,),
            in_specs=[
                pl.BlockSpec((_NUM_VIEWS, _PROMPT_LEN, _DIM), lambda i, idx: (0, 0, 0)),
                pl.BlockSpec(memory_space=pltpu.HBM),
            ],
            out_specs=pl.BlockSpec(memory_space=pltpu.HBM),
            scratch_shapes=[
                pltpu.VMEM((_BB, _TAIL, _DIM), jnp.float32),
                pltpu.SemaphoreType.DMA,
            ],
        ),
        out_shape=jax.ShapeDtypeStruct((_BATCH, _PROMPT_LEN, _DIM), jnp.float32),
        input_output_aliases={2: 0},
    )(view_id, prompts, sc_out)


def kernel(view_id, prompts):
    idx = view_id.astype(jnp.int32)
    table_sc = prompts[:, :_PLEN_SC, :]
    sc_out = _sc_gather(idx, table_sc)
    return _tc_tail(idx, prompts, sc_out)
